# baseline (speedup 1.0000x reference)
import functools

import jax
import jax.numpy as jnp
from jax import lax
from jax.experimental import pallas as pl
from jax.experimental.pallas import tpu as pltpu
from jax.experimental.pallas import tpu_sc as plsc

VOCAB = 1000000
D = 32
B = 4096
L_SEQ = 50
NC = 2
NS = 16
NW = NC * NS
BPW = B // NW


VBLK = 98304
VGRID = -(-VOCAB // VBLK)


def _matvec_body(w_ref, embt_ref, p_ref):
    p_ref[...] = jnp.dot(w_ref[...], embt_ref[...],
                         preferred_element_type=jnp.float32)[0]


_matvec = pl.pallas_call(
    _matvec_body,
    grid=(VGRID,),
    in_specs=[
        pl.BlockSpec((1, D), lambda i: (0, 0)),
        pl.BlockSpec((D, VBLK), lambda i: (0, i)),
    ],
    out_specs=pl.BlockSpec((VBLK,), lambda i: (i,)),
    out_shape=jax.ShapeDtypeStruct((VGRID * VBLK,), jnp.float32),
)


def _pool_body(xt_hbm, p_hbm, wb_hbm, out_hbm, idx_v, val_v, wb_v, out_v, sem):
    wid = lax.axis_index("s") * NC + lax.axis_index("c")
    base = wid * BPW

    pltpu.sync_copy(xt_hbm.at[:, pl.ds(base, BPW)], idx_v)
    pltpu.sync_copy(wb_hbm, wb_v)
    bias = wb_v[pl.ds(0, 16)]
    inv_l = jnp.float32(1.0 / L_SEQ)

    for l in range(L_SEQ):
        pltpu.make_async_copy(p_hbm.at[idx_v.at[l]], val_v.at[l], sem).start()

    acc = [jnp.zeros((16,), jnp.float32) for _ in range(BPW // 16)]
    for l in range(L_SEQ):
        pltpu.make_async_copy(p_hbm.at[idx_v.at[l]], val_v.at[l], sem).wait()
        for j in range(BPW // 16):
            acc[j] = acc[j] + val_v[l, pl.ds(j * 16, 16)]

    for j in range(BPW // 16):
        out_v[pl.ds(j * 16, 16)] = acc[j] * inv_l + bias

    pltpu.sync_copy(out_v, out_hbm.at[pl.ds(base, BPW)])


_mesh = plsc.VectorSubcoreMesh(
    core_axis_name="c", subcore_axis_name="s", num_cores=NC, num_subcores=NS)

_pool = functools.partial(
    pl.kernel,
    out_type=jax.ShapeDtypeStruct((B,), jnp.float32),
    mesh=_mesh,
    compiler_params=pltpu.CompilerParams(use_tc_tiling_on_sc=False),
    scratch_types=[
        pltpu.VMEM((L_SEQ, BPW), jnp.int32),
        pltpu.VMEM((L_SEQ, BPW), jnp.float32),
        pltpu.VMEM((16,), jnp.float32),
        pltpu.VMEM((BPW,), jnp.float32),
        pltpu.SemaphoreType.DMA,
    ],
)(_pool_body)


@jax.jit
def kernel(x, emb, W, b):
    p = _matvec(W, emb.T)
    wb = jnp.broadcast_to(b, (16,))
    out = _pool(x.astype(jnp.int32).T, p, wb)
    return out.reshape(B, 1)


# speedup vs baseline: 1.0147x; 1.0147x over previous
import functools

import jax
import jax.numpy as jnp
from jax import lax
from jax.experimental import pallas as pl
from jax.experimental.pallas import tpu as pltpu
from jax.experimental.pallas import tpu_sc as plsc

VOCAB = 1000000
D = 32
B = 4096
L_SEQ = 50
NC = 2
NS = 16
NW = NC * NS
BPW = B // NW


VBLK = 65536
VGRID = -(-VOCAB // VBLK)


def _matvec_body(w_ref, embt_ref, p_ref):
    p_ref[...] = jnp.dot(w_ref[...], embt_ref[...],
                         preferred_element_type=jnp.float32)[0]


_matvec = pl.pallas_call(
    _matvec_body,
    grid=(VGRID,),
    in_specs=[
        pl.BlockSpec((1, D), lambda i: (0, 0)),
        pl.BlockSpec((D, VBLK), lambda i: (0, i)),
    ],
    out_specs=pl.BlockSpec((VBLK,), lambda i: (i,)),
    out_shape=jax.ShapeDtypeStruct((VGRID * VBLK,), jnp.float32),
)


def _pool_body(xt_hbm, p_hbm, wb_hbm, out_hbm, idx_v, val_v, wb_v, out_v, sem):
    wid = lax.axis_index("s") * NC + lax.axis_index("c")
    base = wid * BPW

    pltpu.sync_copy(xt_hbm.at[:, pl.ds(base, BPW)], idx_v)
    pltpu.sync_copy(wb_hbm, wb_v)
    bias = wb_v[pl.ds(0, 16)]
    inv_l = jnp.float32(1.0 / L_SEQ)

    for l in range(L_SEQ):
        pltpu.make_async_copy(p_hbm.at[idx_v.at[l]], val_v.at[l], sem).start()

    acc = [jnp.zeros((16,), jnp.float32) for _ in range(BPW // 16)]
    for l in range(L_SEQ):
        pltpu.make_async_copy(p_hbm.at[idx_v.at[l]], val_v.at[l], sem).wait()
        for j in range(BPW // 16):
            acc[j] = acc[j] + val_v[l, pl.ds(j * 16, 16)]

    for j in range(BPW // 16):
        out_v[pl.ds(j * 16, 16)] = acc[j] * inv_l + bias

    pltpu.sync_copy(out_v, out_hbm.at[pl.ds(base, BPW)])


_mesh = plsc.VectorSubcoreMesh(
    core_axis_name="c", subcore_axis_name="s", num_cores=NC, num_subcores=NS)

_pool = functools.partial(
    pl.kernel,
    out_type=jax.ShapeDtypeStruct((B,), jnp.float32),
    mesh=_mesh,
    compiler_params=pltpu.CompilerParams(use_tc_tiling_on_sc=False),
    scratch_types=[
        pltpu.VMEM((L_SEQ, BPW), jnp.int32),
        pltpu.VMEM((L_SEQ, BPW), jnp.float32),
        pltpu.VMEM((16,), jnp.float32),
        pltpu.VMEM((BPW,), jnp.float32),
        pltpu.SemaphoreType.DMA,
    ],
)(_pool_body)


@jax.jit
def kernel(x, emb, W, b):
    p = _matvec(W, emb.T)
    wb = jnp.broadcast_to(b, (16,))
    out = _pool(x.astype(jnp.int32).T, p, wb)
    return out.reshape(B, 1)
